# initial kernel scaffold (unmeasured)
import jax
import jax.numpy as jnp
from jax import lax
from jax.experimental import pallas as pl
from jax.experimental.pallas import tpu as pltpu


def kernel(
    x,
):
    def body(*refs):
        pass

    out_shape = jax.ShapeDtypeStruct(..., jnp.float32)
    return pl.pallas_call(body, out_shape=out_shape)(...)


# baseline (device time: 314025 ns/iter reference)
import jax
import jax.numpy as jnp
from jax import lax
from jax.experimental import pallas as pl
from jax.experimental.pallas import tpu as pltpu

N_DEV = 4
M_PER = 4096
N_PER = 1024
M_GLOBAL = N_DEV * M_PER


def kernel(x):
    x16 = x.astype(jnp.bfloat16)

    def body(x_ref, out_ref, send_sems, recv_sems, copy_sem):
        my = lax.axis_index("i")

        barrier = pltpu.get_barrier_semaphore()
        for j in range(N_DEV):
            @pl.when(my != j)
            def _():
                pl.semaphore_signal(
                    barrier, inc=1,
                    device_id=(j,), device_id_type=pl.DeviceIdType.MESH,
                )
        pl.semaphore_wait(barrier, N_DEV - 1)

        local = pltpu.make_async_copy(
            x_ref.at[:, pl.ds(my * N_PER, N_PER)],
            out_ref.at[pl.ds(my * M_PER, M_PER), :],
            copy_sem,
        )
        local.start()

        for j in range(N_DEV):
            @pl.when(my != j)
            def _():
                rdma = pltpu.make_async_remote_copy(
                    src_ref=x_ref.at[:, pl.ds(j * N_PER, N_PER)],
                    dst_ref=out_ref.at[pl.ds(my * M_PER, M_PER), :],
                    send_sem=send_sems.at[j],
                    recv_sem=recv_sems.at[my],
                    device_id=(j,),
                    device_id_type=pl.DeviceIdType.MESH,
                )
                rdma.start()

        for j in range(N_DEV):
            @pl.when(my != j)
            def _():
                send_done = pltpu.make_async_remote_copy(
                    src_ref=x_ref.at[:, pl.ds(j * N_PER, N_PER)],
                    dst_ref=out_ref.at[pl.ds(my * M_PER, M_PER), :],
                    send_sem=send_sems.at[j],
                    recv_sem=recv_sems.at[my],
                    device_id=(j,),
                    device_id_type=pl.DeviceIdType.MESH,
                )
                send_done.wait_send()
                recv_done = pltpu.make_async_remote_copy(
                    src_ref=x_ref.at[:, pl.ds(j * N_PER, N_PER)],
                    dst_ref=out_ref.at[pl.ds(j * M_PER, M_PER), :],
                    send_sem=send_sems.at[j],
                    recv_sem=recv_sems.at[j],
                    device_id=(j,),
                    device_id_type=pl.DeviceIdType.MESH,
                )
                recv_done.wait_recv()
        local.wait()

    return pl.pallas_call(
        body,
        out_shape=jax.ShapeDtypeStruct((M_GLOBAL, N_PER), jnp.bfloat16),
        in_specs=[pl.BlockSpec(memory_space=pltpu.MemorySpace.HBM)],
        out_specs=pl.BlockSpec(memory_space=pltpu.MemorySpace.HBM),
        scratch_shapes=[
            pltpu.SemaphoreType.DMA((N_DEV,)),
            pltpu.SemaphoreType.DMA((N_DEV,)),
            pltpu.SemaphoreType.DMA,
        ],
        compiler_params=pltpu.CompilerParams(collective_id=0),
    )(x16)


# device time: 250096 ns/iter; 1.2556x vs baseline; 1.2556x over previous
import jax
import jax.numpy as jnp
from jax import lax
from jax.experimental import pallas as pl
from jax.experimental.pallas import tpu as pltpu

N_DEV = 4
M_PER = 4096
N_PER = 1024
M_GLOBAL = N_DEV * M_PER


def kernel(x):
    x16 = x.astype(jnp.bfloat16)

    def body(x_ref, out_ref, send_sems, recv_sems, copy_sem):
        my = lax.axis_index("i")

        barrier = pltpu.get_barrier_semaphore()
        for j in range(N_DEV):
            @pl.when(my != j)
            def _():
                pl.semaphore_signal(
                    barrier, inc=1,
                    device_id=(j,), device_id_type=pl.DeviceIdType.MESH,
                )
        pl.semaphore_wait(barrier, N_DEV - 1)

        local = pltpu.make_async_copy(
            x_ref.at[:, pl.ds(my * N_PER, N_PER)],
            out_ref.at[pl.ds(my * M_PER, M_PER), :],
            copy_sem,
        )
        local.start()

        for j in range(N_DEV):
            @pl.when(my != j)
            def _():
                rdma = pltpu.make_async_remote_copy(
                    src_ref=x_ref.at[:, pl.ds(j * N_PER, N_PER)],
                    dst_ref=out_ref.at[pl.ds(my * M_PER, M_PER), :],
                    send_sem=send_sems.at[j],
                    recv_sem=recv_sems.at[my],
                    device_id=(j,),
                    device_id_type=pl.DeviceIdType.MESH,
                )
                rdma.start()

        for j in range(N_DEV):
            @pl.when(my != j)
            def _():
                send_done = pltpu.make_async_remote_copy(
                    src_ref=x_ref.at[:, pl.ds(j * N_PER, N_PER)],
                    dst_ref=out_ref.at[pl.ds(my * M_PER, M_PER), :],
                    send_sem=send_sems.at[j],
                    recv_sem=recv_sems.at[my],
                    device_id=(j,),
                    device_id_type=pl.DeviceIdType.MESH,
                )
                send_done.wait_send()
                recv_done = pltpu.make_async_remote_copy(
                    src_ref=x_ref.at[:, pl.ds(j * N_PER, N_PER)],
                    dst_ref=out_ref.at[pl.ds(j * M_PER, M_PER), :],
                    send_sem=send_sems.at[j],
                    recv_sem=recv_sems.at[j],
                    device_id=(j,),
                    device_id_type=pl.DeviceIdType.MESH,
                )
                recv_done.wait_recv()
        local.wait()

    return pl.pallas_call(
        body,
        out_shape=jax.ShapeDtypeStruct((M_GLOBAL, N_PER), jnp.bfloat16),
        in_specs=[pl.BlockSpec(memory_space=pltpu.MemorySpace.VMEM)],
        out_specs=pl.BlockSpec(memory_space=pltpu.MemorySpace.HBM),
        scratch_shapes=[
            pltpu.SemaphoreType.DMA((N_DEV,)),
            pltpu.SemaphoreType.DMA((N_DEV,)),
            pltpu.SemaphoreType.DMA,
        ],
        compiler_params=pltpu.CompilerParams(collective_id=0),
    )(x16)


# device time: 212756 ns/iter; 1.4760x vs baseline; 1.1755x over previous
import jax
import jax.numpy as jnp
from jax import lax
from jax.experimental import pallas as pl
from jax.experimental.pallas import tpu as pltpu

N_DEV = 4
M_PER = 4096
N_PER = 1024
M_GLOBAL = N_DEV * M_PER
CHUNKS = 2
C_ROWS = M_PER // CHUNKS
DST_ORDER = (1, 3, 2, 0)
UNITS = [(d, c) for c in range(CHUNKS) for d in DST_ORDER]


def kernel(x):
    def body(x_ref, out_ref, stage, sendbuf, stage_sems, send_sems,
             recv_sems, copy_sems):
        my = lax.axis_index("i")

        barrier = pltpu.get_barrier_semaphore()
        for j in range(N_DEV):
            @pl.when(my != j)
            def _():
                pl.semaphore_signal(
                    barrier, inc=1,
                    device_id=(j,), device_id_type=pl.DeviceIdType.MESH,
                )
        pl.semaphore_wait(barrier, N_DEV - 1)

        def stage_in(u):
            d, c = UNITS[u]
            j = (my + d) % N_DEV
            cp = pltpu.make_async_copy(
                x_ref.at[pl.ds(c * C_ROWS, C_ROWS), pl.ds(j * N_PER, N_PER)],
                stage.at[u % 2],
                stage_sems.at[u % 2],
            )
            cp.start()
            return cp

        pending = [None] * len(UNITS)
        stage_cp = stage_in(0)
        for u, (d, c) in enumerate(UNITS):
            nxt = stage_in(u + 1) if u + 1 < len(UNITS) else None
            stage_cp.wait()
            sendbuf[u] = stage[u % 2].astype(jnp.bfloat16)
            j = (my + d) % N_DEV
            if d == 0:
                cp = pltpu.make_async_copy(
                    sendbuf.at[u],
                    out_ref.at[pl.ds(my * M_PER + c * C_ROWS, C_ROWS), :],
                    copy_sems.at[c],
                )
                cp.start()
                pending[u] = cp
            else:
                rdma = pltpu.make_async_remote_copy(
                    src_ref=sendbuf.at[u],
                    dst_ref=out_ref.at[pl.ds(my * M_PER + c * C_ROWS, C_ROWS), :],
                    send_sem=send_sems.at[u],
                    recv_sem=recv_sems.at[my, c],
                    device_id=(j,),
                    device_id_type=pl.DeviceIdType.MESH,
                )
                rdma.start()
                pending[u] = rdma
            stage_cp = nxt

        for u, (d, c) in enumerate(UNITS):
            if d == 0:
                pending[u].wait()
            else:
                pending[u].wait_send()
        for j in range(N_DEV):
            for c in range(CHUNKS):
                @pl.when(my != j)
                def _():
                    recv_done = pltpu.make_async_remote_copy(
                        src_ref=sendbuf.at[0],
                        dst_ref=out_ref.at[
                            pl.ds(j * M_PER + c * C_ROWS, C_ROWS), :],
                        send_sem=send_sems.at[0],
                        recv_sem=recv_sems.at[j, c],
                        device_id=(j,),
                        device_id_type=pl.DeviceIdType.MESH,
                    )
                    recv_done.wait_recv()

    n_units = len(UNITS)
    return pl.pallas_call(
        body,
        out_shape=jax.ShapeDtypeStruct((M_GLOBAL, N_PER), jnp.bfloat16),
        in_specs=[pl.BlockSpec(memory_space=pltpu.MemorySpace.HBM)],
        out_specs=pl.BlockSpec(memory_space=pltpu.MemorySpace.HBM),
        scratch_shapes=[
            pltpu.VMEM((2, C_ROWS, N_PER), jnp.float32),
            pltpu.VMEM((n_units, C_ROWS, N_PER), jnp.bfloat16),
            pltpu.SemaphoreType.DMA((2,)),
            pltpu.SemaphoreType.DMA((n_units,)),
            pltpu.SemaphoreType.DMA((N_DEV, CHUNKS)),
            pltpu.SemaphoreType.DMA((CHUNKS,)),
        ],
        compiler_params=pltpu.CompilerParams(
            collective_id=0, vmem_limit_bytes=56 * 1024 * 1024,
        ),
    )(x)


# device time: 209790 ns/iter; 1.4969x vs baseline; 1.0141x over previous
import jax
import jax.numpy as jnp
from jax import lax
from jax.experimental import pallas as pl
from jax.experimental.pallas import tpu as pltpu

N_DEV = 4
M_PER = 4096
N_PER = 1024
M_GLOBAL = N_DEV * M_PER
CHUNKS = 4
C_ROWS = M_PER // CHUNKS
DST_ORDER = (1, 3, 2, 0)
UNITS = [(d, c) for c in range(CHUNKS) for d in DST_ORDER]


def kernel(x):
    def body(x_ref, out_ref, stage, sendbuf, stage_sems, send_sems,
             recv_sems, copy_sems):
        my = lax.axis_index("i")

        barrier = pltpu.get_barrier_semaphore()
        for j in range(N_DEV):
            @pl.when(my != j)
            def _():
                pl.semaphore_signal(
                    barrier, inc=1,
                    device_id=(j,), device_id_type=pl.DeviceIdType.MESH,
                )
        pl.semaphore_wait(barrier, N_DEV - 1)

        def stage_in(u):
            d, c = UNITS[u]
            j = (my + d) % N_DEV
            cp = pltpu.make_async_copy(
                x_ref.at[pl.ds(c * C_ROWS, C_ROWS), pl.ds(j * N_PER, N_PER)],
                stage.at[u % 2],
                stage_sems.at[u % 2],
            )
            cp.start()
            return cp

        pending = [None] * len(UNITS)
        stage_cp = stage_in(0)
        for u, (d, c) in enumerate(UNITS):
            nxt = stage_in(u + 1) if u + 1 < len(UNITS) else None
            stage_cp.wait()
            sendbuf[u] = stage[u % 2].astype(jnp.bfloat16)
            j = (my + d) % N_DEV
            if d == 0:
                cp = pltpu.make_async_copy(
                    sendbuf.at[u],
                    out_ref.at[pl.ds(my * M_PER + c * C_ROWS, C_ROWS), :],
                    copy_sems.at[c],
                )
                cp.start()
                pending[u] = cp
            else:
                rdma = pltpu.make_async_remote_copy(
                    src_ref=sendbuf.at[u],
                    dst_ref=out_ref.at[pl.ds(my * M_PER + c * C_ROWS, C_ROWS), :],
                    send_sem=send_sems.at[u],
                    recv_sem=recv_sems.at[my, c],
                    device_id=(j,),
                    device_id_type=pl.DeviceIdType.MESH,
                )
                rdma.start()
                pending[u] = rdma
            stage_cp = nxt

        for u, (d, c) in enumerate(UNITS):
            if d == 0:
                pending[u].wait()
            else:
                pending[u].wait_send()
        for j in range(N_DEV):
            for c in range(CHUNKS):
                @pl.when(my != j)
                def _():
                    recv_done = pltpu.make_async_remote_copy(
                        src_ref=sendbuf.at[0],
                        dst_ref=out_ref.at[
                            pl.ds(j * M_PER + c * C_ROWS, C_ROWS), :],
                        send_sem=send_sems.at[0],
                        recv_sem=recv_sems.at[j, c],
                        device_id=(j,),
                        device_id_type=pl.DeviceIdType.MESH,
                    )
                    recv_done.wait_recv()

    n_units = len(UNITS)
    return pl.pallas_call(
        body,
        out_shape=jax.ShapeDtypeStruct((M_GLOBAL, N_PER), jnp.bfloat16),
        in_specs=[pl.BlockSpec(memory_space=pltpu.MemorySpace.HBM)],
        out_specs=pl.BlockSpec(memory_space=pltpu.MemorySpace.HBM),
        scratch_shapes=[
            pltpu.VMEM((2, C_ROWS, N_PER), jnp.float32),
            pltpu.VMEM((n_units, C_ROWS, N_PER), jnp.bfloat16),
            pltpu.SemaphoreType.DMA((2,)),
            pltpu.SemaphoreType.DMA((n_units,)),
            pltpu.SemaphoreType.DMA((N_DEV, CHUNKS)),
            pltpu.SemaphoreType.DMA((CHUNKS,)),
        ],
        compiler_params=pltpu.CompilerParams(
            collective_id=0, vmem_limit_bytes=56 * 1024 * 1024,
        ),
    )(x)
